# baseline (speedup 1.0000x reference)
import jax
import jax.numpy as jnp
from jax import lax
from jax.experimental import pallas as pl
from jax.experimental.pallas import tpu as pltpu
from jax.experimental.pallas import tpu_sc as plsc

_DIM = 32
_NE = 8192
_N = 8192
_BR1 = 512
_BR2 = 512
_NC = 2
_NS = 16
_NW = _NC * _NS
_RPW = _N // _NW
_L = 16


def _dist_body(x_ref, w_ref, d_ref, idx_ref, w2c, w2x):
    r = pl.program_id(0)

    @pl.when(r == 0)
    def _():
        wt = w_ref[...]
        w2x[...] = wt + wt
        w2c[...] = jnp.sum(wt * wt, axis=1)[None, :]

    x = x_ref[...]
    x2 = jnp.sum(x * x, axis=1, keepdims=True)
    mm2 = jax.lax.dot_general(x, w2x[...], (((1,), (1,)), ((), ())),
                              preferred_element_type=jnp.float32)
    d = (x2 - mm2) + w2c[...]
    d_ref[...] = d
    rmin = jnp.min(d, axis=1, keepdims=True)
    col = jax.lax.broadcasted_iota(jnp.int32, d.shape, 1)
    idx_ref[...] = jnp.min(jnp.where(d == rmin, col, jnp.int32(2**30)),
                           axis=1, keepdims=True)


def _sc_body(idx_hbm, w_hbm, q_hbm, idx_v, rows_v, sem):
    wid = lax.axis_index("s") * _NC + lax.axis_index("c")
    base = wid * _RPW
    pltpu.sync_copy(idx_hbm.at[pl.ds(base, _RPW)], idx_v)
    pltpu.async_copy(w_hbm.at[idx_v], rows_v, sem).wait()
    pltpu.sync_copy(rows_v, q_hbm.at[pl.ds(base, _RPW)])


def _sc_call(idx1, w):
    mesh = plsc.VectorSubcoreMesh(core_axis_name="c", subcore_axis_name="s")
    f = pl.kernel(
        _sc_body,
        out_type=jax.ShapeDtypeStruct((_N, _DIM), jnp.float32),
        mesh=mesh,
        compiler_params=pltpu.CompilerParams(use_tc_tiling_on_sc=False),
        scratch_types=[
            pltpu.VMEM((_RPW,), jnp.int32),
            pltpu.VMEM((_RPW, _DIM), jnp.float32),
            pltpu.SemaphoreType.DMA,
        ],
    )
    return f(idx1, w)


def _enc_body(idx_ref, q_ref, x_ref, enc_ref, qst_ref, loss_ref, perp_ref,
              hist, sse):
    r = pl.program_id(0)
    nr = pl.num_programs(0)
    idx = idx_ref[...]
    col = jax.lax.broadcasted_iota(jnp.int32, (_BR2, _NE), 1)
    enc = (col == idx).astype(jnp.float32)
    enc_ref[...] = enc
    colsum = jnp.sum(enc, axis=0, keepdims=True)

    @pl.when(r == 0)
    def _():
        hist[...] = colsum

    @pl.when(r > 0)
    def _():
        hist[...] += colsum

    xt = x_ref[...]
    diff = q_ref[...] - xt
    qst_ref[...] = xt + diff
    tile_sse = jnp.sum(diff * diff)
    prev = jnp.where(r == 0, 0.0, sse[0, 0])
    sse[0, 0] = prev + tile_sse

    @pl.when(r == nr - 1)
    def _():
        loss_ref[0, 0] = sse[0, 0] * (1.25 / float(_N * _DIM))
        avg = hist[...] * (1.0 / float(_N))
        ent = jnp.sum(avg * jnp.log(avg + 1e-10))
        perp_ref[0, 0] = jnp.exp(-ent)


def kernel(inputs, w):
    x = jnp.transpose(inputs, (0, 2, 3, 1))
    input_shape = x.shape
    flat = x.reshape(_N, _DIM)

    dist, idx = pl.pallas_call(
        _dist_body,
        grid=(_N // _BR1,),
        in_specs=[
            pl.BlockSpec((_BR1, _DIM), lambda r: (r, 0)),
            pl.BlockSpec((_NE, _DIM), lambda r: (0, 0)),
        ],
        out_specs=[
            pl.BlockSpec((_BR1, _NE), lambda r: (r, 0)),
            pl.BlockSpec((_BR1, 1), lambda r: (r, 0)),
        ],
        out_shape=[
            jax.ShapeDtypeStruct((_N, _NE), jnp.float32),
            jax.ShapeDtypeStruct((_N, 1), jnp.int32),
        ],
        scratch_shapes=[
            pltpu.VMEM((1, _NE), jnp.float32),
            pltpu.VMEM((_NE, _DIM), jnp.float32),
        ],
    )(flat, w)

    q0 = _sc_call(idx.reshape(_N), w)

    enc, q, loss, perp = pl.pallas_call(
        _enc_body,
        grid=(_N // _BR2,),
        in_specs=[
            pl.BlockSpec((_BR2, 1), lambda r: (r, 0)),
            pl.BlockSpec((_BR2, _DIM), lambda r: (r, 0)),
            pl.BlockSpec((_BR2, _DIM), lambda r: (r, 0)),
        ],
        out_specs=[
            pl.BlockSpec((_BR2, _NE), lambda r: (r, 0)),
            pl.BlockSpec((_BR2, _DIM), lambda r: (r, 0)),
            pl.BlockSpec((1, 1), lambda r: (0, 0),
                         memory_space=pltpu.SMEM),
            pl.BlockSpec((1, 1), lambda r: (0, 0),
                         memory_space=pltpu.SMEM),
        ],
        out_shape=[
            jax.ShapeDtypeStruct((_N, _NE), jnp.float32),
            jax.ShapeDtypeStruct((_N, _DIM), jnp.float32),
            jax.ShapeDtypeStruct((1, 1), jnp.float32),
            jax.ShapeDtypeStruct((1, 1), jnp.float32),
        ],
        scratch_shapes=[
            pltpu.VMEM((1, _NE), jnp.float32),
            pltpu.SMEM((1, 1), jnp.float32),
        ],
    )(idx, q0, flat)

    quantized = jnp.transpose(q.reshape(input_shape), (0, 3, 1, 2))
    return (dist, quantized, loss[0, 0], enc, idx, perp[0, 0])


# speedup vs baseline: 1.0151x; 1.0151x over previous
import jax
import jax.numpy as jnp
from jax import lax
from jax.experimental import pallas as pl
from jax.experimental.pallas import tpu as pltpu
from jax.experimental.pallas import tpu_sc as plsc

_DIM = 32
_NE = 8192
_N = 8192
_BR1 = 512
_BR2 = 512
_NC = 2
_NS = 16
_NW = _NC * _NS
_RPW = _N // _NW
_L = 16


def _dist_body(x_ref, w_ref, d_ref, idx_ref, w2c, w2x):
    r = pl.program_id(0)

    @pl.when(r == 0)
    def _():
        wt = w_ref[...]
        w2x[...] = wt + wt
        w2c[...] = jnp.sum(wt * wt, axis=1)[None, :]

    x = x_ref[...]
    x2 = jnp.sum(x * x, axis=1, keepdims=True)
    mm2 = jax.lax.dot_general(x, w2x[...], (((1,), (1,)), ((), ())),
                              preferred_element_type=jnp.float32)
    d = (x2 - mm2) + w2c[...]
    d_ref[...] = d
    rmin = jnp.min(d, axis=1, keepdims=True)
    col = jax.lax.broadcasted_iota(jnp.int32, d.shape, 1)
    idx_ref[...] = jnp.min(jnp.where(d == rmin, col, jnp.int32(2**30)),
                           axis=1, keepdims=True)


def _sc_body(idx_hbm, w_hbm, x_hbm, q_hbm, part_hbm,
             idx_v, rows_v, x_v, psum_v, sem):
    wid = lax.axis_index("s") * _NC + lax.axis_index("c")
    base = wid * _RPW

    pltpu.sync_copy(idx_hbm.at[pl.ds(base, _RPW)], idx_v)
    cp = pltpu.async_copy(w_hbm.at[idx_v], rows_v, sem)
    pltpu.sync_copy(x_hbm.at[pl.ds(base, _RPW)], x_v)
    cp.wait()

    def _st(i, acc):
        for off in (0, _L):
            xc = x_v[i, pl.ds(off, _L)]
            qc = rows_v[i, pl.ds(off, _L)]
            dd = qc - xc
            rows_v[i, pl.ds(off, _L)] = xc + dd
            acc = acc + dd * dd
        return acc
    acc = lax.fori_loop(0, _RPW, _st, jnp.zeros((_L,), jnp.float32))
    psum_v[...] = acc
    pltpu.sync_copy(rows_v, q_hbm.at[pl.ds(base, _RPW)])
    pltpu.sync_copy(psum_v, part_hbm.at[wid])


def _sc_call(idx1, w, flat):
    mesh = plsc.VectorSubcoreMesh(core_axis_name="c", subcore_axis_name="s")
    f = pl.kernel(
        _sc_body,
        out_type=[
            jax.ShapeDtypeStruct((_N, _DIM), jnp.float32),
            jax.ShapeDtypeStruct((_NW, _L), jnp.float32),
        ],
        mesh=mesh,
        compiler_params=pltpu.CompilerParams(use_tc_tiling_on_sc=False),
        scratch_types=[
            pltpu.VMEM((_RPW,), jnp.int32),
            pltpu.VMEM((_RPW, _DIM), jnp.float32),
            pltpu.VMEM((_RPW, _DIM), jnp.float32),
            pltpu.VMEM((_L,), jnp.float32),
            pltpu.SemaphoreType.DMA,
        ],
    )
    return f(idx1, w, flat)


def _enc_body(idx_ref, part_ref, enc_ref, loss_ref, perp_ref, hist):
    r = pl.program_id(0)
    nr = pl.num_programs(0)
    idx = idx_ref[...]
    col = jax.lax.broadcasted_iota(jnp.int32, (_BR2, _NE), 1)
    enc = (col == idx).astype(jnp.float32)
    enc_ref[...] = enc
    colsum = jnp.sum(enc, axis=0, keepdims=True)

    @pl.when(r == 0)
    def _():
        hist[...] = colsum

    @pl.when(r > 0)
    def _():
        hist[...] += colsum

    @pl.when(r == nr - 1)
    def _():
        loss_ref[0, 0] = jnp.sum(part_ref[...]) * (1.25 / float(_N * _DIM))
        avg = hist[...] * (1.0 / float(_N))
        ent = jnp.sum(avg * jnp.log(avg + 1e-10))
        perp_ref[0, 0] = jnp.exp(-ent)


def kernel(inputs, w):
    x = jnp.transpose(inputs, (0, 2, 3, 1))
    input_shape = x.shape
    flat = x.reshape(_N, _DIM)

    dist, idx = pl.pallas_call(
        _dist_body,
        grid=(_N // _BR1,),
        in_specs=[
            pl.BlockSpec((_BR1, _DIM), lambda r: (r, 0)),
            pl.BlockSpec((_NE, _DIM), lambda r: (0, 0)),
        ],
        out_specs=[
            pl.BlockSpec((_BR1, _NE), lambda r: (r, 0)),
            pl.BlockSpec((_BR1, 1), lambda r: (r, 0)),
        ],
        out_shape=[
            jax.ShapeDtypeStruct((_N, _NE), jnp.float32),
            jax.ShapeDtypeStruct((_N, 1), jnp.int32),
        ],
        scratch_shapes=[
            pltpu.VMEM((1, _NE), jnp.float32),
            pltpu.VMEM((_NE, _DIM), jnp.float32),
        ],
    )(flat, w)

    q, parts = _sc_call(idx.reshape(_N), w, flat)

    enc, loss, perp = pl.pallas_call(
        _enc_body,
        grid=(_N // _BR2,),
        in_specs=[
            pl.BlockSpec((_BR2, 1), lambda r: (r, 0)),
            pl.BlockSpec((_NW, _L), lambda r: (0, 0)),
        ],
        out_specs=[
            pl.BlockSpec((_BR2, _NE), lambda r: (r, 0)),
            pl.BlockSpec((1, 1), lambda r: (0, 0),
                         memory_space=pltpu.SMEM),
            pl.BlockSpec((1, 1), lambda r: (0, 0),
                         memory_space=pltpu.SMEM),
        ],
        out_shape=[
            jax.ShapeDtypeStruct((_N, _NE), jnp.float32),
            jax.ShapeDtypeStruct((1, 1), jnp.float32),
            jax.ShapeDtypeStruct((1, 1), jnp.float32),
        ],
        scratch_shapes=[
            pltpu.VMEM((1, _NE), jnp.float32),
        ],
    )(idx, parts)

    quantized = jnp.transpose(q.reshape(input_shape), (0, 3, 1, 2))
    return (dist, quantized, loss[0, 0], enc, idx, perp[0, 0])
